# baseline (speedup 1.0000x reference)
import functools

import jax
import jax.numpy as jnp
from jax import lax
from jax.experimental import pallas as pl
from jax.experimental.pallas import tpu as pltpu
from jax.experimental.pallas import tpu_sc as plsc

N = 8192
D = 256
L = 16
NC = 2
NS = 16
ROWS_PER_W = N // (NC * NS)
BIG_I = 2**30

_mesh = plsc.VectorSubcoreMesh(core_axis_name="c", subcore_axis_name="s")


@functools.partial(
    pl.kernel,
    out_type=[
        jax.ShapeDtypeStruct((NC, L), jnp.float32),
        jax.ShapeDtypeStruct((NC, D), jnp.float32),
    ],
    mesh=_mesh,
    compiler_params=pltpu.CompilerParams(needs_layout_passes=False),
    scratch_types=[
        pltpu.VMEM((D,), jnp.float32),
        pltpu.VMEM((ROWS_PER_W, D), jnp.float32),
        pltpu.VMEM((2 * L,), jnp.float32),
        pltpu.VMEM((L, 2 * L), jnp.float32),
        pltpu.VMEM((L,), jnp.float32),
        pltpu.VMEM((D,), jnp.float32),
        pltpu.VMEM_SHARED((L, 2 * L), jnp.float32),
    ],
)
def _nn_kernel(u_hbm, loc_hbm, val_hbm, d_out, row_out,
               u_v, loc_v, tmp, buf, tmp_d, row_v, shared):
    c = lax.axis_index("c")
    s = lax.axis_index("s")
    base = (c * NS + s) * ROWS_PER_W

    pltpu.sync_copy(u_hbm, u_v)
    pltpu.sync_copy(loc_hbm.at[pl.ds(base, ROWS_PER_W)], loc_v)

    u_regs = [u_v[pl.ds(L * j, L)] for j in range(D // L)]

    def row_body(r, carry):
        best_d, best_r = carry
        acc = jnp.zeros((L,), jnp.float32)
        for j in range(D // L):
            dlt = loc_v[r, pl.ds(L * j, L)] - u_regs[j]
            acc = acc + dlt * dlt
        dist = jnp.sum(acc)
        better = dist < best_d
        best_d = jnp.where(better, dist, best_d)
        best_r = jnp.where(better, r, best_r)
        return best_d, best_r

    best_d, best_r = lax.fori_loop(
        0, ROWS_PER_W, row_body,
        (jnp.float32(jnp.inf), jnp.int32(0)), unroll=4)
    gidx = best_r + base

    lane = lax.iota(jnp.int32, L)
    tmp[pl.ds(0, L)] = jnp.where(lane == s, best_d, jnp.float32(jnp.inf))
    tmp[pl.ds(L, L)] = lax.bitcast_convert_type(
        jnp.where(lane == s, gidx, BIG_I), jnp.float32)
    pltpu.sync_copy(tmp, shared.at[s])
    plsc.subcore_barrier()

    @pl.when(s == 0)
    def _merge():
        pltpu.sync_copy(shared, buf)
        dv = buf[0, pl.ds(0, L)]
        iv = lax.bitcast_convert_type(buf[0, pl.ds(L, L)], jnp.int32)
        for j in range(1, NS):
            dj = buf[j, pl.ds(0, L)]
            ij = lax.bitcast_convert_type(buf[j, pl.ds(L, L)], jnp.int32)
            m = dj < dv
            iv = jnp.where(m, ij, iv)
            dv = jnp.where(m, dj, dv)
        mind = jnp.min(dv)
        idx = jnp.min(jnp.where(dv == mind, iv, BIG_I))
        pltpu.sync_copy(val_hbm.at[idx], row_v)
        pltpu.sync_copy(row_v, row_out.at[c])
        tmp_d[...] = jnp.broadcast_to(mind, (L,))
        pltpu.sync_copy(tmp_d, d_out.at[c])


def kernel(u, loc, val, p):
    del p
    d_part, rows = _nn_kernel(u, loc, val)
    return jnp.where(d_part[0, 0] <= d_part[1, 0], rows[0], rows[1])


# speedup vs baseline: 3.1339x; 3.1339x over previous
import functools

import jax
import jax.numpy as jnp
from jax import lax
from jax.experimental import pallas as pl
from jax.experimental.pallas import tpu as pltpu

N = 8192
D = 256
BR = 1024
NB = N // BR
BIG_I = 2**30


def _nn_body(u_ref, loc_ref, val_ref, out_ref, best_d, best_i, row_v, sem):
    i = pl.program_id(0)

    @pl.when(i == 0)
    def _init():
        best_d[0] = jnp.float32(jnp.inf)
        best_i[0] = jnp.int32(0)

    d = loc_ref[...] - u_ref[...]
    s = jnp.sum(d * d, axis=1)
    m = jnp.min(s)
    iota = lax.iota(jnp.int32, BR)
    li = jnp.min(jnp.where(s == m, iota, BIG_I))

    better = m < best_d[0]

    @pl.when(better)
    def _update():
        best_d[0] = m
        best_i[0] = i * BR + li

    @pl.when(i == NB - 1)
    def _gather():
        copy = pltpu.make_async_copy(val_ref.at[best_i[0]], row_v, sem)
        copy.start()
        copy.wait()
        out_ref[...] = row_v[...]


@functools.partial(jax.jit, static_argnames=())
def _nn(u2, loc, val):
    return pl.pallas_call(
        _nn_body,
        grid=(NB,),
        in_specs=[
            pl.BlockSpec((1, D), lambda i: (0, 0)),
            pl.BlockSpec((BR, D), lambda i: (i, 0)),
            pl.BlockSpec(memory_space=pl.ANY),
        ],
        out_specs=pl.BlockSpec(memory_space=pltpu.VMEM),
        out_shape=jax.ShapeDtypeStruct((D,), jnp.float32),
        scratch_shapes=[
            pltpu.SMEM((1,), jnp.float32),
            pltpu.SMEM((1,), jnp.int32),
            pltpu.VMEM((D,), jnp.float32),
            pltpu.SemaphoreType.DMA,
        ],
    )(u2, loc, val)


def kernel(u, loc, val, p):
    del p
    return _nn(u.reshape(1, D), loc, val)


# speedup vs baseline: 3.9010x; 1.2448x over previous
import functools

import jax
import jax.numpy as jnp
from jax import lax
from jax.experimental import pallas as pl
from jax.experimental.pallas import tpu as pltpu

N = 8192
D = 256
BR = 1024
NB = N // BR
BIG_I = 2**30

_DN = (((1,), (1,)), ((), ()))


def _nn_body(u_ref, loc_ref, val_ref, out_ref, best_d, best_i, row_v, sem):
    i = pl.program_id(0)

    d = loc_ref[...] - u_ref[...]
    d2 = d * d
    ones = jnp.ones((8, D), jnp.float32)
    s = lax.dot_general(ones, d2, _DN, preferred_element_type=jnp.float32)
    gi = i * BR + lax.broadcasted_iota(jnp.int32, (8, BR), 1)

    @pl.when(i == 0)
    def _init():
        best_d[...] = s
        best_i[...] = gi

    @pl.when(i > 0)
    def _update():
        mask = s < best_d[...]
        best_d[...] = jnp.where(mask, s, best_d[...])
        best_i[...] = jnp.where(mask, gi, best_i[...])

    @pl.when(i == NB - 1)
    def _gather():
        m = jnp.min(best_d[...])
        idx = jnp.min(jnp.where(best_d[...] == m, best_i[...], BIG_I))
        copy = pltpu.make_async_copy(val_ref.at[idx], row_v, sem)
        copy.start()
        copy.wait()
        out_ref[...] = row_v[...]


@functools.partial(jax.jit, static_argnames=())
def _nn(u2, loc, val):
    return pl.pallas_call(
        _nn_body,
        grid=(NB,),
        in_specs=[
            pl.BlockSpec((1, D), lambda i: (0, 0)),
            pl.BlockSpec((BR, D), lambda i: (i, 0)),
            pl.BlockSpec(memory_space=pl.ANY),
        ],
        out_specs=pl.BlockSpec(memory_space=pltpu.VMEM),
        out_shape=jax.ShapeDtypeStruct((D,), jnp.float32),
        scratch_shapes=[
            pltpu.VMEM((8, BR), jnp.float32),
            pltpu.VMEM((8, BR), jnp.int32),
            pltpu.VMEM((D,), jnp.float32),
            pltpu.SemaphoreType.DMA,
        ],
    )(u2, loc, val)


def kernel(u, loc, val, p):
    del p
    return _nn(u.reshape(1, D), loc, val)
